# baseline (speedup 1.0000x reference)
import math

import numpy as np
import jax
import jax.numpy as jnp
from jax import lax
from jax.experimental import pallas as pl
from jax.experimental.pallas import tpu as pltpu
from jax.experimental.pallas import tpu_sc as plsc

_EMB = 64
_MAIN = 44
_B, _L = 16384, 50
_NPOS = _B * _L
_NW = 32
_ROWS = 8
_CHUNK = _ROWS * _L
_NCHUNK = _B // (_NW * _ROWS)
_NGRP = _CHUNK // 16
_N_RANKS = 14
_MSTR = 45
_LROWS = 32
_OUTW = _ROWS * _LROWS * 128
_RELAY_ROWS = 64


def _body(pk_h, main_h, quad_h, cos_h, sin_h, cmb_h,
          out_h, mask_h,
          pk_v0, pk_v1,
          out_v0, out_v1, mask_v0, mask_v1,
          main_t, quad_t, cos_t, sin_t, cmb_v,
          hist, shist,
          sem_in0, sem_in1, sem_out0, sem_out1):
    wid = lax.axis_index("s") * 2 + lax.axis_index("c")
    sems_in = (sem_in0, sem_in1)
    sems_out = (sem_out0, sem_out1)
    pks = (pk_v0, pk_v1)
    outs = (out_v0, out_v1)
    masks = (mask_v0, mask_v1)

    pltpu.sync_copy(main_h, main_t)
    pltpu.sync_copy(quad_h, quad_t)
    pltpu.sync_copy(cos_h, cos_t)
    pltpu.sync_copy(sin_h, sin_t)
    pltpu.sync_copy(cmb_h, cmb_v)

    iota = lax.iota(jnp.int32, 16)
    ones = jnp.ones((16,), jnp.float32)
    zf = jnp.zeros((16,), jnp.float32)

    def zero_body(k, carry):
        zi = k * 16 + iota
        plsc.store_scatter(out_v0, [zi], zf)
        plsc.store_scatter(out_v1, [zi], zf)
        return carry

    lax.fori_loop(0, _OUTW // 16, zero_body, 0)

    def issue_in(c, b):
        base = (wid * _NCHUNK + c) * _CHUNK
        pltpu.async_copy(pk_h.at[pl.ds(base, _CHUNK)], pks[b], sems_in[b])

    def wait_in(b):
        pltpu.make_async_copy(pk_h.at[pl.ds(0, _CHUNK)], pks[b],
                              sems_in[b]).wait()

    def issue_out(c, b):
        base = (wid * _NCHUNK + c) * _CHUNK
        pltpu.async_copy(outs[b],
                         out_h.at[pl.ds((wid * _NCHUNK + c) * _OUTW, _OUTW)],
                         sems_out[b])
        pltpu.async_copy(masks[b], mask_h.at[pl.ds(base, _CHUNK)],
                         sems_out[b])

    def wait_out(b):
        pltpu.make_async_copy(outs[b],
                              out_h.at[pl.ds(0, _OUTW)],
                              sems_out[b]).wait()
        pltpu.make_async_copy(masks[b], mask_h.at[pl.ds(0, _CHUNK)],
                              sems_out[b]).wait()

    def compute(b):
        pk_v = pks[b]
        out_v = outs[b]
        mask_v = masks[b]

        for k in range(16):
            hist[pl.ds(k * 16, 16)] = zf
        for k in range(8):
            shist[pl.ds(k * 16, 16)] = zf

        def hist_body(g, h_carry):
            s = g * 16
            pk = pk_v[pl.ds(s, 16)]
            rk = (pk >> 24) & 15
            st = (pk >> 21) & 7
            rid = cmb_v[pl.ds(s, 16)] >> 16
            plsc.addupdate_scatter(hist, [rid * 32 + rk], ones)
            plsc.addupdate_scatter(shist, [rid * 16 + st], ones)
            return h_carry

        lax.fori_loop(0, _NGRP, hist_body, 0)

        def grp_body(g, g_carry):
            s = g * 16
            pk = pk_v[pl.ds(s, 16)]
            idxv = pk & 255
            env = (pk >> 8) & 15
            edv = (pk >> 12) & 7
            slv = (pk >> 15) & 7
            sgv = (pk >> 18) & 7
            st = (pk >> 21) & 7
            rk = (pk >> 24) & 15
            cmb = cmb_v[pl.ds(s, 16)]
            rid = cmb >> 16
            pb = cmb & 0xFFFF
            hb = rid * 32
            sr = plsc.load_gather(hist, [hb + rk])
            sr = jnp.where(rk == 0, zf, sr)
            ss = plsc.load_gather(shist, [rid * 16 + st])
            ss = jnp.where(st == 0, zf, ss)
            fl = jnp.where(ss >= 5.0, ones, zf)
            up = plsc.load_gather(hist, [jnp.maximum(hb + rk - 1, 0)])
            up = jnp.where(rk == 0, zf, up)
            dn = plsc.load_gather(hist, [hb + rk + 1])
            cs = plsc.load_gather(cos_t, [rk])
            sn = plsc.load_gather(sin_t, [rk])
            mask_v[pl.ds(s, 16)] = ((idxv == 0) & (rk == 0)).astype(jnp.int32)

            feats = {37: cs, 38: sn, 39: up, 40: dn, 41: fl, 42: ss, 43: sr}
            m = idxv * _MSTR
            col = iota - iota
            one_i = col + 1
            for d in range(_MAIN):
                v = plsc.load_gather(main_t, [m])
                f = feats.get(d)
                if f is not None:
                    v = v + f
                plsc.store_scatter(out_v, [pb + col], v)
                m = m + 1
                col = col + one_i
            plsc.addupdate_scatter(out_v, [pb + st], ones)
            plsc.addupdate_scatter(out_v, [pb + 5 + rk], ones)

            col = col + 4
            for t, ivec in enumerate((sgv, env, edv, slv)):
                q = ivec * 5 + t * 80
                for k in range(4):
                    v = plsc.load_gather(quad_t, [q])
                    plsc.store_scatter(out_v, [pb + col], v)
                    q = q + 1
                    col = col + one_i
            return g_carry

        lax.fori_loop(0, _NGRP, grp_body, 0)

    issue_in(0, 0)

    def pair_body(p, carry):
        c = 2 * p
        issue_in(c + 1, 1)
        wait_in(0)

        @pl.when(p >= 1)
        def _():
            wait_out(0)

        compute(0)
        issue_out(c, 0)

        @pl.when(p <= _NCHUNK // 2 - 2)
        def _():
            issue_in(c + 2, 0)

        wait_in(1)

        @pl.when(p >= 1)
        def _():
            wait_out(1)

        compute(1)
        issue_out(c + 1, 1)
        return carry

    lax.fori_loop(0, _NCHUNK // 2, pair_body, 0)
    wait_out(0)
    wait_out(1)


def _pad_rows(t, n):
    return jnp.zeros((n, t.shape[1]), t.dtype).at[: t.shape[0]].set(t)


def kernel(indices, enhancement, edition, seal, debuffed, segment, suit, rank,
           scalar_properties, general_index_table, enhancement_table,
           edition_table, seal_table, segment_table, debuffed_table,
           suit_table, rank_table):
    del debuffed, debuffed_table, suit_table, rank_table
    packed = (indices | (enhancement << 8) | (edition << 12) | (seal << 15)
              | (segment << 18) | (suit << 21) | (rank << 24))
    pk_f = packed.reshape(-1)

    ph = (np.arange(16, dtype=np.float64) + 1.0) * math.pi / _N_RANKS
    cos_t = jnp.asarray(np.cos(ph), jnp.float32)
    sin_t = jnp.asarray(np.sin(ph), jnp.float32)
    p = np.arange(_CHUNK)
    r, l = p // _L, p % _L
    m = l % 25
    h = (m + l // 25) & 1
    offw = (r * _LROWS + m) * 128 + h * _EMB
    cmb_t = jnp.asarray((r << 16) | offw, jnp.int32)

    def _pad5(t):
        q = _pad_rows(t, 16)
        return jnp.pad(q, ((0, 0), (0, 1))).reshape(-1)

    quad = jnp.concatenate([
        _pad5(segment_table), _pad5(enhancement_table),
        _pad5(edition_table), _pad5(seal_table),
    ])
    main_flat = jnp.pad(general_index_table,
                        ((0, 0), (0, _MSTR - _MAIN))).reshape(-1)

    mesh = plsc.VectorSubcoreMesh(core_axis_name="c", subcore_axis_name="s")
    out, mask = pl.kernel(
        _body,
        out_type=(
            jax.ShapeDtypeStruct((_B * _LROWS * 128,), jnp.float32),
            jax.ShapeDtypeStruct((_NPOS,), jnp.int32),
        ),
        mesh=mesh,
        compiler_params=pltpu.CompilerParams(needs_layout_passes=False),
        scratch_types=(
            (pltpu.VMEM((_CHUNK,), jnp.int32),) * 2
            + (pltpu.VMEM((_OUTW,), jnp.float32),) * 2
            + (pltpu.VMEM((_CHUNK,), jnp.int32),) * 2
            + (
                pltpu.VMEM((160 * _MSTR,), jnp.float32),
                pltpu.VMEM((320,), jnp.float32),
                pltpu.VMEM((16,), jnp.float32),
                pltpu.VMEM((16,), jnp.float32),
                pltpu.VMEM((_CHUNK,), jnp.int32),
                pltpu.VMEM((256,), jnp.float32),
                pltpu.VMEM((128,), jnp.float32),
                pltpu.SemaphoreType.DMA,
                pltpu.SemaphoreType.DMA,
                pltpu.SemaphoreType.DMA,
                pltpu.SemaphoreType.DMA,
            )
        ),
    )(pk_f, main_flat, quad, cos_t, sin_t, cmb_t)

    def _relayout(in_ref, scal_ref, out_ref):
        x = in_ref[...].reshape(_RELAY_ROWS, _LROWS, 128)
        lo = x[:, :25, :_EMB]
        hi = x[:, :25, _EMB:]
        even = lax.broadcasted_iota(jnp.int32, (1, 25, 1), 1) % 2 == 0
        y = jnp.concatenate(
            [jnp.where(even, lo, hi), jnp.where(even, hi, lo)], axis=1)
        s = scal_ref[...]
        out_ref[...] = jnp.concatenate(
            [y[:, :, :_MAIN], s, y[:, :, _MAIN + 4:]], axis=2)

    embeddings = pl.pallas_call(
        _relayout,
        grid=(_B // _RELAY_ROWS,),
        in_specs=[
            pl.BlockSpec((_RELAY_ROWS * _LROWS * 128,), lambda i: (i,)),
            pl.BlockSpec((_RELAY_ROWS, _L, 4), lambda i: (i, 0, 0)),
        ],
        out_specs=pl.BlockSpec((_RELAY_ROWS, _L, _EMB), lambda i: (i, 0, 0)),
        out_shape=jax.ShapeDtypeStruct((_B, _L, _EMB), jnp.float32),
    )(out, scalar_properties.astype(jnp.float32))
    padding_mask = mask.reshape(_B, _L).astype(bool)
    return embeddings, padding_mask


# speedup vs baseline: 1.0175x; 1.0175x over previous
import math

import numpy as np
import jax
import jax.numpy as jnp
from jax import lax
from jax.experimental import pallas as pl
from jax.experimental.pallas import tpu as pltpu
from jax.experimental.pallas import tpu_sc as plsc

_EMB = 64
_MAIN = 44
_B, _L = 16384, 50
_NPOS = _B * _L
_NW = 32
_ROWS = 8
_CHUNK = _ROWS * _L
_NCHUNK = _B // (_NW * _ROWS)
_NGRP = _CHUNK // 16
_N_RANKS = 14
_MSTR = 45
_LROWS = 32
_OUTW = _ROWS * _LROWS * 128
_RELAY_ROWS = 64


def _body(pk_h, main_h, quad_h, cos_h, sin_h, cmb_h,
          out_h, mask_h,
          pk_v0, pk_v1,
          out_v0, out_v1, mask_v0, mask_v1,
          main_t, quad_t, cos_t, sin_t, cmb_v,
          hist, shist,
          sem_in0, sem_in1, sem_out0, sem_out1):
    wid = lax.axis_index("s") * 2 + lax.axis_index("c")
    sems_in = (sem_in0, sem_in1)
    sems_out = (sem_out0, sem_out1)
    pks = (pk_v0, pk_v1)
    outs = (out_v0, out_v1)
    masks = (mask_v0, mask_v1)

    pltpu.sync_copy(main_h, main_t)
    pltpu.sync_copy(quad_h, quad_t)
    pltpu.sync_copy(cos_h, cos_t)
    pltpu.sync_copy(sin_h, sin_t)
    pltpu.sync_copy(cmb_h, cmb_v)

    iota = lax.iota(jnp.int32, 16)
    ones = jnp.ones((16,), jnp.float32)
    zf = jnp.zeros((16,), jnp.float32)

    def zero_body(k, carry):
        zi = k * 16 + iota
        plsc.store_scatter(out_v0, [zi], zf)
        plsc.store_scatter(out_v1, [zi], zf)
        return carry

    lax.fori_loop(0, _OUTW // 16, zero_body, 0)

    def issue_in(c, b):
        base = (wid * _NCHUNK + c) * _CHUNK
        pltpu.async_copy(pk_h.at[pl.ds(base, _CHUNK)], pks[b], sems_in[b])

    def wait_in(b):
        pltpu.make_async_copy(pk_h.at[pl.ds(0, _CHUNK)], pks[b],
                              sems_in[b]).wait()

    def issue_out(c, b):
        base = (wid * _NCHUNK + c) * _CHUNK
        pltpu.async_copy(outs[b],
                         out_h.at[pl.ds((wid * _NCHUNK + c) * _OUTW, _OUTW)],
                         sems_out[b])
        pltpu.async_copy(masks[b], mask_h.at[pl.ds(base, _CHUNK)],
                         sems_out[b])

    def wait_out(b):
        pltpu.make_async_copy(outs[b],
                              out_h.at[pl.ds(0, _OUTW)],
                              sems_out[b]).wait()
        pltpu.make_async_copy(masks[b], mask_h.at[pl.ds(0, _CHUNK)],
                              sems_out[b]).wait()

    def compute(b):
        pk_v = pks[b]
        out_v = outs[b]
        mask_v = masks[b]

        for k in range(16):
            hist[pl.ds(k * 16, 16)] = zf
        for k in range(8):
            shist[pl.ds(k * 16, 16)] = zf

        def hist_body(g, h_carry):
            s = g * 16
            pk = pk_v[pl.ds(s, 16)]
            rk = (pk >> 24) & 15
            st = (pk >> 21) & 7
            rid = cmb_v[pl.ds(s, 16)] >> 16
            plsc.addupdate_scatter(hist, [rid * 32 + rk], ones)
            plsc.addupdate_scatter(shist, [rid * 16 + st], ones)
            return h_carry

        lax.fori_loop(0, _NGRP, hist_body, 0)

        def grp_body(g, g_carry):
            s = g * 16
            pk = pk_v[pl.ds(s, 16)]
            idxv = pk & 255
            env = (pk >> 8) & 15
            edv = (pk >> 12) & 7
            slv = (pk >> 15) & 7
            sgv = (pk >> 18) & 7
            st = (pk >> 21) & 7
            rk = (pk >> 24) & 15
            cmb = cmb_v[pl.ds(s, 16)]
            rid = cmb >> 16
            pb = cmb & 0xFFFF
            hb = rid * 32
            sr = plsc.load_gather(hist, [hb + rk])
            sr = jnp.where(rk == 0, zf, sr)
            ss = plsc.load_gather(shist, [rid * 16 + st])
            ss = jnp.where(st == 0, zf, ss)
            fl = jnp.where(ss >= 5.0, ones, zf)
            up = plsc.load_gather(hist, [jnp.maximum(hb + rk - 1, 0)])
            up = jnp.where(rk == 0, zf, up)
            dn = plsc.load_gather(hist, [hb + rk + 1])
            cs = plsc.load_gather(cos_t, [rk])
            sn = plsc.load_gather(sin_t, [rk])
            mask_v[pl.ds(s, 16)] = ((idxv == 0) & (rk == 0)).astype(jnp.int32)

            feats = {37: cs, 38: sn, 39: up, 40: dn, 41: fl, 42: ss, 43: sr}
            m = idxv * _MSTR
            col = iota - iota
            one_i = col + 1
            for d in range(_MAIN):
                v = plsc.load_gather(main_t, [m])
                f = feats.get(d)
                if f is not None:
                    v = v + f
                plsc.store_scatter(out_v, [pb + col], v)
                m = m + 1
                col = col + one_i
            plsc.addupdate_scatter(out_v, [pb + st], ones)
            plsc.addupdate_scatter(out_v, [pb + 5 + rk], ones)

            col = col + 4
            for t, ivec in enumerate((sgv, env, edv, slv)):
                q = ivec * 5 + t * 80
                for k in range(4):
                    v = plsc.load_gather(quad_t, [q])
                    plsc.store_scatter(out_v, [pb + col], v)
                    q = q + 1
                    col = col + one_i
            return g_carry

        lax.fori_loop(0, _NGRP, grp_body, 0)

    issue_in(0, 0)

    def pair_body(p, carry):
        c = 2 * p
        issue_in(c + 1, 1)
        wait_in(0)

        @pl.when(p >= 1)
        def _():
            wait_out(0)

        compute(0)
        issue_out(c, 0)

        @pl.when(p <= _NCHUNK // 2 - 2)
        def _():
            issue_in(c + 2, 0)

        wait_in(1)

        @pl.when(p >= 1)
        def _():
            wait_out(1)

        compute(1)
        issue_out(c + 1, 1)
        return carry

    lax.fori_loop(0, _NCHUNK // 2, pair_body, 0)
    wait_out(0)
    wait_out(1)


def _pad_rows(t, n):
    return jnp.zeros((n, t.shape[1]), t.dtype).at[: t.shape[0]].set(t)


def kernel(indices, enhancement, edition, seal, debuffed, segment, suit, rank,
           scalar_properties, general_index_table, enhancement_table,
           edition_table, seal_table, segment_table, debuffed_table,
           suit_table, rank_table):
    del debuffed, debuffed_table, suit_table, rank_table
    packed = (indices | (enhancement << 8) | (edition << 12) | (seal << 15)
              | (segment << 18) | (suit << 21) | (rank << 24))
    pk_f = packed.reshape(-1)

    ph = (np.arange(16, dtype=np.float64) + 1.0) * math.pi / _N_RANKS
    cos_t = jnp.asarray(np.cos(ph), jnp.float32)
    sin_t = jnp.asarray(np.sin(ph), jnp.float32)
    p = np.arange(_CHUNK)
    r, l = p // _L, p % _L
    offw = (r * _LROWS + l % 25) * 128 + (l // 25) * _EMB
    cmb_t = jnp.asarray((r << 16) | offw, jnp.int32)

    def _pad5(t):
        q = _pad_rows(t, 16)
        return jnp.pad(q, ((0, 0), (0, 1))).reshape(-1)

    quad = jnp.concatenate([
        _pad5(segment_table), _pad5(enhancement_table),
        _pad5(edition_table), _pad5(seal_table),
    ])
    main_flat = jnp.pad(general_index_table,
                        ((0, 0), (0, _MSTR - _MAIN))).reshape(-1)

    mesh = plsc.VectorSubcoreMesh(core_axis_name="c", subcore_axis_name="s")
    out, mask = pl.kernel(
        _body,
        out_type=(
            jax.ShapeDtypeStruct((_B * _LROWS * 128,), jnp.float32),
            jax.ShapeDtypeStruct((_NPOS,), jnp.int32),
        ),
        mesh=mesh,
        compiler_params=pltpu.CompilerParams(needs_layout_passes=False),
        scratch_types=(
            (pltpu.VMEM((_CHUNK,), jnp.int32),) * 2
            + (pltpu.VMEM((_OUTW,), jnp.float32),) * 2
            + (pltpu.VMEM((_CHUNK,), jnp.int32),) * 2
            + (
                pltpu.VMEM((160 * _MSTR,), jnp.float32),
                pltpu.VMEM((320,), jnp.float32),
                pltpu.VMEM((16,), jnp.float32),
                pltpu.VMEM((16,), jnp.float32),
                pltpu.VMEM((_CHUNK,), jnp.int32),
                pltpu.VMEM((256,), jnp.float32),
                pltpu.VMEM((128,), jnp.float32),
                pltpu.SemaphoreType.DMA,
                pltpu.SemaphoreType.DMA,
                pltpu.SemaphoreType.DMA,
                pltpu.SemaphoreType.DMA,
            )
        ),
    )(pk_f, main_flat, quad, cos_t, sin_t, cmb_t)

    def _relayout(in_ref, scal_ref, out_ref):
        x = in_ref[...].reshape(_RELAY_ROWS, _LROWS, 128)
        lo = x[:, :25, :_EMB]
        hi = x[:, :25, _EMB:]
        y = jnp.concatenate([lo, hi], axis=1)
        s = scal_ref[...]
        out_ref[...] = jnp.concatenate(
            [y[:, :, :_MAIN], s, y[:, :, _MAIN + 4:]], axis=2)

    embeddings = pl.pallas_call(
        _relayout,
        grid=(_B // _RELAY_ROWS,),
        in_specs=[
            pl.BlockSpec((_RELAY_ROWS * _LROWS * 128,), lambda i: (i,)),
            pl.BlockSpec((_RELAY_ROWS, _L, 4), lambda i: (i, 0, 0)),
        ],
        out_specs=pl.BlockSpec((_RELAY_ROWS, _L, _EMB), lambda i: (i, 0, 0)),
        out_shape=jax.ShapeDtypeStruct((_B, _L, _EMB), jnp.float32),
    )(out, scalar_properties.astype(jnp.float32))
    padding_mask = mask.reshape(_B, _L).astype(bool)
    return embeddings, padding_mask
